# baseline (speedup 1.0000x reference)
import functools

import jax
import jax.numpy as jnp
from jax import lax
from jax.experimental import pallas as pl
from jax.experimental.pallas import tpu as pltpu
from jax.experimental.pallas import tpu_sc as plsc

B = 16384
K = 200
NC = 2
NS = 16
L = 16
NW = NC * NS
PER_W = B // NW
C = 16
NCHUNK = PER_W // C
GROUPS = C // L
NSLOT = 6


def _sc_body(ent_hbm, rel_hbm, eet_hbm, rri_hbm,
             out_hbm, ent_v, rel_v, out_v, bufs, sems):
    wid = lax.axis_index("s") * NC + lax.axis_index("c")
    base = wid * PER_W

    pltpu.sync_copy(ent_hbm.at[pl.ds(base * 2, 2 * PER_W)], ent_v)
    pltpu.sync_copy(rel_hbm.at[pl.ds(base, PER_W)], rel_v)

    def start(c):
        slot = c % NSLOT
        ei = ent_v.at[pl.ds(c * 2 * C, 2 * C)]
        re = rel_v.at[pl.ds(c * C, C)]
        ent_b, rel_b = bufs[slot]
        sem = sems[slot]
        return [
            pltpu.async_copy(eet_hbm.at[ei], ent_b, sem),
            pltpu.async_copy(rri_hbm.at[re], rel_b, sem),
        ]

    lane = lax.iota(jnp.int32, L)
    zero = jnp.zeros((L,), jnp.float32)

    def compute(c):
        slot = c % NSLOT
        ent_b, rel_b = bufs[slot]
        for g in range(GROUPS):
            rows = lane + (g * L)
            rows_t = rows + C

            def kbody(k, carry):
                a1, a2 = carry
                cols = jnp.full((L,), 0, jnp.int32) + k
                cols2 = cols + K
                hh = plsc.load_gather(ent_b, [rows, cols])
                ht = plsc.load_gather(ent_b, [rows, cols2])
                th = plsc.load_gather(ent_b, [rows_t, cols])
                tt = plsc.load_gather(ent_b, [rows_t, cols2])
                rv = plsc.load_gather(rel_b, [rows, cols])
                riv = plsc.load_gather(rel_b, [rows, cols2])
                return a1 + hh * rv * tt, a2 + th * riv * ht

            a1, a2 = lax.fori_loop(0, K, kbody, (zero, zero), unroll=4)
            score = jnp.clip((a1 + a2) * 0.5, -20.0, 20.0)
            out_v[pl.ds(c * C + g * L, L)] = score

    cps = {}
    for c in range(min(NSLOT, NCHUNK)):
        cps[c] = start(c)
    for c in range(NCHUNK):
        for cp in cps.pop(c):
            cp.wait()
        compute(c)
        if c + NSLOT < NCHUNK:
            cps[c + NSLOT] = start(c + NSLOT)

    pltpu.sync_copy(out_v, out_hbm.at[pl.ds(base, PER_W)])


@functools.cache
def _build():
    mesh = plsc.VectorSubcoreMesh(
        core_axis_name="c", subcore_axis_name="s", num_cores=NC,
        num_subcores=NS)
    slot = lambda: [
        pltpu.VMEM((2 * C, 2 * K), jnp.float32),
        pltpu.VMEM((C, 2 * K), jnp.float32),
    ]
    scratch = [
        pltpu.VMEM((2 * PER_W,), jnp.int32),
        pltpu.VMEM((PER_W,), jnp.int32),
        pltpu.VMEM((PER_W,), jnp.float32),
        [slot() for _ in range(NSLOT)],
        [pltpu.SemaphoreType.DMA for _ in range(NSLOT)],
    ]
    return pl.kernel(
        _sc_body,
        out_type=jax.ShapeDtypeStruct((B,), jnp.float32),
        mesh=mesh,
        scratch_types=scratch,
        compiler_params=pltpu.CompilerParams(
            use_tc_tiling_on_sc=False, needs_layout_passes=False),
    )


@jax.jit
def kernel(head, rel, tail, embed_eh, embed_et, embed_r, embed_ri):
    head = head.astype(jnp.int32)
    rel = rel.astype(jnp.int32)
    tail = tail.astype(jnp.int32)
    ent = jnp.stack(
        [head.reshape(NW, NCHUNK, C), tail.reshape(NW, NCHUNK, C)],
        axis=2).reshape(2 * B)
    eet = jnp.concatenate([embed_eh, embed_et], axis=1)
    rri = jnp.concatenate([embed_r, embed_ri], axis=1)
    return _build()(ent, rel, eet, rri)
